# baseline (speedup 1.0000x reference)
import math

import jax
import jax.numpy as jnp
from jax.experimental import pallas as pl

B, N1, N2 = 4, 4096, 4096
TI = 256
NBLK = N1 // TI


def _acos01(x):
    p = jnp.float32(-0.0012624911)
    for c in (0.0066700901, -0.0170881256, 0.0308918810, -0.0501743046,
              0.0889789874, -0.2145988016, 1.5707963050):
        p = p * x + jnp.float32(c)
    return p * jnp.sqrt(jnp.maximum(1.0 - x, 0.0))


def _chamfer_body(x1_ref, nr_ref, x2_ref, cat6_ref, pt_ref, nm_ref, dist_ref, ang_ref):
    x1 = x1_ref[0, 0]
    x2 = x2_ref[0]

    dots = jax.lax.dot_general(
        x1, x2, (((1,), (1,)), ((), ())), preferred_element_type=jnp.float32
    )
    r0 = cat6_ref[0, 0:1, :]
    r1 = cat6_ref[0, 1:2, :]
    r2 = cat6_ref[0, 2:3, :]
    sq1 = jnp.sum(x1 * x1, axis=1, keepdims=True)
    sq2 = (r0 * r0 + r1 * r1) + r2 * r2
    d = sq1 + sq2 - 2.0 * dots

    dmin = jnp.min(d, axis=1, keepdims=True)
    jidx = jax.lax.broadcasted_iota(jnp.int32, (TI, N2), 1)
    idx = jnp.min(jnp.where(d <= dmin, jidx, N2), axis=1, keepdims=True)

    sel = jidx == idx
    zero = jnp.zeros((), jnp.float32)

    def pick(k):
        row = cat6_ref[0, k:k + 1, :]
        return jnp.sum(jnp.where(sel, row, zero), axis=1, keepdims=True)

    p0, p1, p2 = pick(0), pick(1), pick(2)
    g0, g1, g2 = pick(3), pick(4), pick(5)

    x10, x11, x12 = x1[:, 0:1], x1[:, 1:2], x1[:, 2:3]
    d0, d1, d2 = x10 - p0, x11 - p1, x12 - p2
    point_dist = d0 * d0 + d1 * d1 + d2 * d2

    nr = nr_ref[0, 0]
    m0, m1, m2 = nr[:, 0:1], nr[:, 1:2], nr[:, 2:3]
    n1n = jnp.sqrt(m0 * m0 + m1 * m1 + m2 * m2)
    inv1 = 1.0 / jnp.maximum(n1n, 1e-12)
    n2n = jnp.sqrt(g0 * g0 + g1 * g1 + g2 * g2)
    inv2 = 1.0 / jnp.maximum(n2n, 1e-12)
    dotn = m0 * g0 + m1 * g1 + m2 * g2
    cosang = jnp.abs(dotn * inv1 * inv2)
    angle = _acos01(jnp.clip(cosang, 0.0, 1.0)) * (180.0 / math.pi)

    pt_ref[0, 0] = jnp.concatenate([p0, p1, p2], axis=1)
    nm_ref[0, 0] = jnp.concatenate([g0, g1, g2], axis=1)
    dist_ref[0, 0] = point_dist
    ang_ref[0, 0] = angle


def kernel(xyz1, xyz2, normal_rebuild, normal_gt):
    x1r = xyz1.reshape(B, NBLK, TI, 3)
    nrr = normal_rebuild.reshape(B, NBLK, TI, 3)
    cat6 = jnp.concatenate(
        [jnp.transpose(xyz2, (0, 2, 1)), jnp.transpose(normal_gt, (0, 2, 1))], axis=1
    )

    grid = (B, NBLK)
    out_shapes = (
        jax.ShapeDtypeStruct((B, NBLK, TI, 3), jnp.float32),
        jax.ShapeDtypeStruct((B, NBLK, TI, 3), jnp.float32),
        jax.ShapeDtypeStruct((B, NBLK, TI, 1), jnp.float32),
        jax.ShapeDtypeStruct((B, NBLK, TI, 1), jnp.float32),
    )
    in_specs = [
        pl.BlockSpec((1, 1, TI, 3), lambda b, i: (b, i, 0, 0)),
        pl.BlockSpec((1, 1, TI, 3), lambda b, i: (b, i, 0, 0)),
        pl.BlockSpec((1, N2, 3), lambda b, i: (b, 0, 0)),
        pl.BlockSpec((1, 6, N2), lambda b, i: (b, 0, 0)),
    ]
    out_specs = (
        pl.BlockSpec((1, 1, TI, 3), lambda b, i: (b, i, 0, 0)),
        pl.BlockSpec((1, 1, TI, 3), lambda b, i: (b, i, 0, 0)),
        pl.BlockSpec((1, 1, TI, 1), lambda b, i: (b, i, 0, 0)),
        pl.BlockSpec((1, 1, TI, 1), lambda b, i: (b, i, 0, 0)),
    )
    pts, nms, dist, ang = pl.pallas_call(
        _chamfer_body,
        grid=grid,
        in_specs=in_specs,
        out_specs=out_specs,
        out_shape=out_shapes,
    )(x1r, nrr, xyz2, cat6)

    return (
        pts.reshape(B, N1, 3),
        nms.reshape(B, N1, 3),
        dist.reshape(B, N1),
        ang.reshape(B, N1),
    )


# speedup vs baseline: 1.8766x; 1.8766x over previous
import functools
import math

import jax
import jax.numpy as jnp
from jax import lax
from jax.experimental import pallas as pl
from jax.experimental.pallas import tpu as pltpu
from jax.experimental.pallas import tpu_sc as plsc

B, N1, N2 = 4, 4096, 4096
TI = 256
NBLK = N1 // TI
BN1 = B * N1


def _acos01(x):
    p = jnp.float32(-0.0012624911)
    for c in (0.0066700901, -0.0170881256, 0.0308918810, -0.0501743046,
              0.0889789874, -0.2145988016, 1.5707963050):
        p = p * x + jnp.float32(c)
    return p * jnp.sqrt(jnp.maximum(1.0 - x, 0.0))


def _nn_body(x1_ref, x2_ref, x2t_ref, idx_ref):
    b = pl.program_id(0)
    x1 = x1_ref[0, 0]
    x2 = x2_ref[0]

    dots = jax.lax.dot_general(
        x1, x2, (((1,), (1,)), ((), ())), preferred_element_type=jnp.float32
    )
    r0 = x2t_ref[0, 0:1, :]
    r1 = x2t_ref[0, 1:2, :]
    r2 = x2t_ref[0, 2:3, :]
    sq1 = jnp.sum(x1 * x1, axis=1, keepdims=True)
    sq2 = (r0 * r0 + r1 * r1) + r2 * r2
    d = sq1 + sq2 - 2.0 * dots

    dmin = jnp.min(d, axis=1, keepdims=True)
    jidx = lax.broadcasted_iota(jnp.int32, (TI, N2), 1)
    idx = jnp.min(jnp.where(d <= dmin, jidx, N2), axis=1, keepdims=True)
    idx_ref[0, 0] = idx + b * N2


def _finish_body(x1t_ref, nrt_ref, g8_ref, dist_ref, ang_ref):
    x10 = x1t_ref[0:1, :]
    x11 = x1t_ref[1:2, :]
    x12 = x1t_ref[2:3, :]
    p0 = g8_ref[0:1, :]
    p1 = g8_ref[1:2, :]
    p2 = g8_ref[2:3, :]
    g0 = g8_ref[3:4, :]
    g1 = g8_ref[4:5, :]
    g2 = g8_ref[5:6, :]

    d0, d1, d2 = x10 - p0, x11 - p1, x12 - p2
    dist_ref[0:1, :] = d0 * d0 + d1 * d1 + d2 * d2

    m0 = nrt_ref[0:1, :]
    m1 = nrt_ref[1:2, :]
    m2 = nrt_ref[2:3, :]
    n1n = jnp.sqrt(m0 * m0 + m1 * m1 + m2 * m2)
    inv1 = 1.0 / jnp.maximum(n1n, 1e-12)
    n2n = jnp.sqrt(g0 * g0 + g1 * g1 + g2 * g2)
    inv2 = 1.0 / jnp.maximum(n2n, 1e-12)
    dotn = m0 * g0 + m1 * g1 + m2 * g2
    cosang = jnp.abs(dotn * inv1 * inv2)
    ang_ref[0:1, :] = _acos01(jnp.clip(cosang, 0.0, 1.0)) * (180.0 / math.pi)


def _make_sc_gather():
    info = plsc.get_sparse_core_info()
    NC, NS, L = info.num_cores, info.num_subcores, info.num_lanes
    NW = NC * NS
    b_per_w = BN1 // NW
    nchunk = b_per_w // L
    mesh = plsc.VectorSubcoreMesh(core_axis_name="c", subcore_axis_name="s")

    @functools.partial(
        pl.kernel, mesh=mesh,
        compiler_params=pltpu.CompilerParams(needs_layout_passes=False),
        out_type=jax.ShapeDtypeStruct((6, BN1), jnp.float32),
        scratch_types=[
            pltpu.VMEM((B * N2 * 6,), jnp.float32),
            pltpu.VMEM((b_per_w,), jnp.int32),
            pltpu.VMEM((8, b_per_w), jnp.float32),
        ],
    )
    def gather_k(table_hbm, idx_hbm, out_hbm, table_v, idx_v, out_v):
        wid = lax.axis_index("s") * NC + lax.axis_index("c")
        base = wid * b_per_w
        pltpu.sync_copy(table_hbm, table_v)
        pltpu.sync_copy(idx_hbm.at[pl.ds(base, b_per_w)], idx_v)
        for c in range(nchunk):
            iv = idx_v[pl.ds(c * L, L)]
            fv = iv * 6
            for k in range(6):
                val = plsc.load_gather(table_v, [fv + k])
                out_v[k, pl.ds(c * L, L)] = val
        for k in range(6):
            pltpu.sync_copy(out_v.at[k], out_hbm.at[k, pl.ds(base, b_per_w)])

    return gather_k


_sc_gather = _make_sc_gather()


def kernel(xyz1, xyz2, normal_rebuild, normal_gt):
    x1r = xyz1.reshape(B, NBLK, TI, 3)
    x2t = jnp.transpose(xyz2, (0, 2, 1))

    idxg = pl.pallas_call(
        _nn_body,
        grid=(B, NBLK),
        in_specs=[
            pl.BlockSpec((1, 1, TI, 3), lambda b, i: (b, i, 0, 0)),
            pl.BlockSpec((1, N2, 3), lambda b, i: (b, 0, 0)),
            pl.BlockSpec((1, 3, N2), lambda b, i: (b, 0, 0)),
        ],
        out_specs=pl.BlockSpec((1, 1, TI, 1), lambda b, i: (b, i, 0, 0)),
        out_shape=jax.ShapeDtypeStruct((B, NBLK, TI, 1), jnp.int32),
    )(x1r, xyz2, x2t)

    table = jnp.concatenate([xyz2, normal_gt], axis=2).reshape(B * N2 * 6)
    g6 = _sc_gather(table, idxg.reshape(BN1))

    x1t = jnp.transpose(xyz1.reshape(BN1, 3), (1, 0))
    nrt = jnp.transpose(normal_rebuild.reshape(BN1, 3), (1, 0))

    dist, ang = pl.pallas_call(
        _finish_body,
        grid=(1,),
        in_specs=[
            pl.BlockSpec((3, BN1), lambda i: (0, 0)),
            pl.BlockSpec((3, BN1), lambda i: (0, 0)),
            pl.BlockSpec((6, BN1), lambda i: (0, 0)),
        ],
        out_specs=(
            pl.BlockSpec((1, BN1), lambda i: (0, 0)),
            pl.BlockSpec((1, BN1), lambda i: (0, 0)),
        ),
        out_shape=(
            jax.ShapeDtypeStruct((1, BN1), jnp.float32),
            jax.ShapeDtypeStruct((1, BN1), jnp.float32),
        ),
    )(x1t, nrt, g6)

    return (
        jnp.transpose(g6[0:3, :], (1, 0)).reshape(B, N1, 3),
        jnp.transpose(g6[3:6, :], (1, 0)).reshape(B, N1, 3),
        dist.reshape(B, N1),
        ang.reshape(B, N1),
    )
